# Initial kernel scaffold; baseline (speedup 1.0000x reference)
import jax
import jax.numpy as jnp
from jax.experimental import pallas as pl


def kernel(x, edge_index, W_l0, W_r0, b0, W_l1, W_r1, b1, W_l2, W_r2, b2):
    raise NotImplementedError("write your pallas kernel here")


# speedup vs baseline: 8.5615x; 8.5615x over previous
import jax
import jax.numpy as jnp
from jax import lax
from jax.experimental import pallas as pl
from jax.experimental.pallas import tpu as pltpu
from jax.experimental.pallas import tpu_sc as plsc

NC = 2
NS = 16
NW = NC * NS

LANES = 128
CR = 2


def _sc_mesh():
    return plsc.VectorSubcoreMesh(core_axis_name="c", subcore_axis_name="s",
                                  num_cores=NC, num_subcores=NS)


def _make_agg_kernel(n_pad, d, rows_per_worker):
    n_chunks = rows_per_worker // CR
    rpt = n_pad // NS

    def body(x_hbm, src_hbm, dst_hbm, out_hbm, agg_sh, zbuf, src_v, dst_v,
             rows_v, gsem):
        c = lax.axis_index("c")
        s = lax.axis_index("s")
        wid = s * NC + c

        zv = jnp.zeros((16,), jnp.float32)
        for i in range(16):
            for j in range(d // 16):
                zbuf[i, pl.ds(j * 16, 16)] = zv
        rbase = s * rpt
        for r in range(rpt // 16):
            pltpu.sync_copy(zbuf, agg_sh.at[pl.ds(rbase + r * 16, 16)])
        plsc.subcore_barrier()

        row0 = wid * rows_per_worker

        def chunk(i, carry):
            base = row0 + i * CR
            pltpu.sync_copy(src_hbm.at[pl.ds(base, CR)], src_v)
            pltpu.sync_copy(dst_hbm.at[pl.ds(base, CR)], dst_v)
            cps = []
            for r in range(CR):
                cp = pltpu.make_async_copy(
                    x_hbm.at[src_v.at[r]],
                    rows_v.at[pl.ds(r * LANES, LANES)], gsem)
                cp.start()
                cps.append(cp)
            for r in range(CR):
                cps[r].wait()
                pltpu.sync_copy(rows_v.at[pl.ds(r * LANES, LANES)],
                                agg_sh.at[dst_v.at[r]], add=True)
            return carry

        lax.fori_loop(0, n_chunks, chunk, 0)
        plsc.subcore_barrier()

        pltpu.sync_copy(agg_sh.at[pl.ds(rbase, rpt)],
                        out_hbm.at[c].at[pl.ds(rbase, rpt)])

    return pl.kernel(
        body,
        out_type=jax.ShapeDtypeStruct((NC, n_pad, d), jnp.float32),
        mesh=_sc_mesh(),
        scratch_types=[
            pltpu.VMEM_SHARED((n_pad, d), jnp.float32),
            pltpu.VMEM((16, d), jnp.float32),
            pltpu.VMEM((CR, LANES), jnp.int32),
            pltpu.VMEM((CR, LANES), jnp.int32),
            pltpu.VMEM((CR * LANES, d), jnp.float32),
            pltpu.SemaphoreType.DMA,
        ],
    )


def _make_deg_kernel(n_pad, rows_per_worker):
    n_chunks = rows_per_worker // CR
    rpt = n_pad // NS

    def body(dst_hbm, out_hbm, deg_sh, zbuf, ones_v, dst_v, gsem):
        del gsem
        c = lax.axis_index("c")
        s = lax.axis_index("s")
        wid = s * NC + c

        zv = jnp.zeros((16,), jnp.float32)
        ov = jnp.ones((16,), jnp.float32)
        for j in range(LANES // 16):
            zbuf[pl.ds(j * 16, 16)] = zv
            ones_v[pl.ds(j * 16, 16)] = ov
        rbase = s * rpt
        for r in range(rpt // LANES):
            pltpu.sync_copy(zbuf, deg_sh.at[pl.ds(rbase + r * LANES, LANES)])
        plsc.subcore_barrier()

        row0 = wid * rows_per_worker

        def chunk(i, carry):
            base = row0 + i * CR
            pltpu.sync_copy(dst_hbm.at[pl.ds(base, CR)], dst_v)
            for r in range(CR):
                pltpu.sync_copy(ones_v, deg_sh.at[dst_v.at[r]], add=True)
            return carry

        lax.fori_loop(0, n_chunks, chunk, 0)
        plsc.subcore_barrier()

        pltpu.sync_copy(deg_sh.at[pl.ds(rbase, rpt)],
                        out_hbm.at[c].at[pl.ds(rbase, rpt)])

    return pl.kernel(
        body,
        out_type=jax.ShapeDtypeStruct((NC, n_pad), jnp.float32),
        mesh=_sc_mesh(),
        scratch_types=[
            pltpu.VMEM_SHARED((n_pad,), jnp.float32),
            pltpu.VMEM((LANES,), jnp.float32),
            pltpu.VMEM((LANES,), jnp.float32),
            pltpu.VMEM((CR, LANES), jnp.int32),
            pltpu.SemaphoreType.DMA,
        ],
    )


def _tc_layer(parts, deg0, deg1, x, wl, wr, b, relu):
    n, d = x.shape
    rb = 1000
    grid = (n // rb,)

    def body(a0, a1, d0, d1, xr, wlr, wrr, br, out):
        deg = jnp.maximum(d0[...] + d1[...], 1.0)
        m = (a0[0] + a1[0]) / deg
        h = jnp.dot(m, wlr[...], preferred_element_type=jnp.float32)
        h = h + jnp.dot(xr[...], wrr[...], preferred_element_type=jnp.float32)
        h = h + br[...][None, :]
        if relu:
            h = jnp.maximum(h, 0.0)
        out[...] = h + xr[...]

    return pl.pallas_call(
        body,
        grid=grid,
        in_specs=[
            pl.BlockSpec((1, rb, d), lambda i: (0, i, 0)),
            pl.BlockSpec((1, rb, d), lambda i: (1, i, 0)),
            pl.BlockSpec((rb, 1), lambda i: (i, 0)),
            pl.BlockSpec((rb, 1), lambda i: (i, 0)),
            pl.BlockSpec((rb, d), lambda i: (i, 0)),
            pl.BlockSpec((d, d), lambda i: (0, 0)),
            pl.BlockSpec((d, d), lambda i: (0, 0)),
            pl.BlockSpec((d,), lambda i: (0,)),
        ],
        out_specs=pl.BlockSpec((rb, d), lambda i: (i, 0)),
        out_shape=jax.ShapeDtypeStruct((n, d), jnp.float32),
    )(parts, parts, deg0, deg1, x, wl, wr, b)


def kernel(x, edge_index, W_l0, W_r0, b0, W_l1, W_r1, b1, W_l2, W_r2, b2):
    n, d = x.shape
    e = edge_index.shape[1]

    n_pad = ((n + 16 * NS - 1) // (16 * NS)) * (16 * NS)
    chunk_edges = NW * LANES * CR
    e_pad = ((e + chunk_edges - 1) // chunk_edges) * chunk_edges
    epw = e_pad // NW
    rows_per_worker = epw // LANES
    pad = e_pad - e

    src = edge_index[0]
    dst = edge_index[1]
    if pad:
        pad_src = (jnp.arange(pad, dtype=jnp.int32) * 97) % n
        pad_dst = n + (jnp.arange(pad, dtype=jnp.int32) % (n_pad - n))
        src = jnp.concatenate([src, pad_src])
        dst = jnp.concatenate([dst, pad_dst.astype(jnp.int32)])
    src2d = src.reshape(-1, LANES)
    dst2d = dst.reshape(-1, LANES)

    deg_k = _make_deg_kernel(n_pad, rows_per_worker)
    agg_k = _make_agg_kernel(n_pad, d, rows_per_worker)

    deg_parts = deg_k(dst2d)
    x, deg_parts = lax.optimization_barrier((x, deg_parts))
    deg0 = lax.slice(deg_parts[0], (0,), (n,)).reshape(n, 1)
    deg1 = lax.slice(deg_parts[1], (0,), (n,)).reshape(n, 1)

    layers = [(W_l0, W_r0, b0), (W_l1, W_r1, b1), (W_l2, W_r2, b2)]
    for i, (wl, wr, b) in enumerate(layers):
        parts = agg_k(x, src2d, dst2d)
        parts, x = lax.optimization_barrier((parts, x))
        parts = lax.slice(parts, (0, 0, 0), (NC, n, d))
        x = _tc_layer(parts, deg0, deg1, x, wl, wr, b, relu=(i != 2))
        x, src2d, dst2d = lax.optimization_barrier((x, src2d, dst2d))
    return x


# speedup vs baseline: 10.4760x; 1.2236x over previous
import jax
import jax.numpy as jnp
from jax import lax
from jax.experimental import pallas as pl
from jax.experimental.pallas import tpu as pltpu
from jax.experimental.pallas import tpu_sc as plsc

NC = 2
NS = 16
NW = NC * NS

LANES = 128
CR = 2


def _sc_mesh():
    return plsc.VectorSubcoreMesh(core_axis_name="c", subcore_axis_name="s",
                                  num_cores=NC, num_subcores=NS)


def _make_agg_kernel(n_pad, d, rows_per_worker):
    n_chunks = rows_per_worker // CR
    rpt = n_pad // NS

    def body(x_hbm, src_hbm, dst_hbm, out_hbm, agg_sh, zbuf, src_v, dst_v,
             rows_v, gsem, isem, zsem):
        c = lax.axis_index("c")
        s = lax.axis_index("s")
        wid = s * NC + c

        zv = jnp.zeros((16,), jnp.float32)
        for i in range(16):
            for j in range(d // 16):
                zbuf[i, pl.ds(j * 16, 16)] = zv
        rbase = s * rpt
        zcps = []
        for r in range(rpt // 16):
            cp = pltpu.make_async_copy(
                zbuf, agg_sh.at[pl.ds(rbase + r * 16, 16)], zsem)
            cp.start()
            zcps.append(cp)

        row0 = wid * rows_per_worker

        def idx_load(j, buf):
            base = row0 + j * CR
            pltpu.make_async_copy(src_hbm.at[pl.ds(base, CR)],
                                  src_v.at[buf], isem).start()
            pltpu.make_async_copy(dst_hbm.at[pl.ds(base, CR)],
                                  dst_v.at[buf], isem).start()

        def idx_wait(j, buf):
            base = row0 + j * CR
            pltpu.make_async_copy(src_hbm.at[pl.ds(base, CR)],
                                  src_v.at[buf], isem).wait()
            pltpu.make_async_copy(dst_hbm.at[pl.ds(base, CR)],
                                  dst_v.at[buf], isem).wait()

        idx_load(0, 0)
        for cp in zcps:
            cp.wait()
        plsc.subcore_barrier()

        def chunk(i, carry):
            cur = lax.rem(i, 2)
            idx_wait(i, cur)
            nxt_j = jnp.minimum(i + 1, n_chunks - 1)
            idx_load(nxt_j, 1 - cur)
            cps = []
            for r in range(CR):
                cp = pltpu.make_async_copy(
                    x_hbm.at[src_v.at[cur].at[r]],
                    rows_v.at[pl.ds(r * LANES, LANES)], gsem)
                cp.start()
                cps.append(cp)
            for r in range(CR):
                cps[r].wait()
                pltpu.sync_copy(rows_v.at[pl.ds(r * LANES, LANES)],
                                agg_sh.at[dst_v.at[cur].at[r]], add=True)
            return carry

        lax.fori_loop(0, n_chunks, chunk, 0)
        idx_wait(n_chunks - 1, lax.rem(n_chunks, 2))
        plsc.subcore_barrier()

        pltpu.sync_copy(agg_sh.at[pl.ds(rbase, rpt)],
                        out_hbm.at[c].at[pl.ds(rbase, rpt)])

    return pl.kernel(
        body,
        out_type=jax.ShapeDtypeStruct((NC, n_pad, d), jnp.float32),
        mesh=_sc_mesh(),
        scratch_types=[
            pltpu.VMEM_SHARED((n_pad, d), jnp.float32),
            pltpu.VMEM((16, d), jnp.float32),
            pltpu.VMEM((2, CR, LANES), jnp.int32),
            pltpu.VMEM((2, CR, LANES), jnp.int32),
            pltpu.VMEM((CR * LANES, d), jnp.float32),
            pltpu.SemaphoreType.DMA,
            pltpu.SemaphoreType.DMA,
            pltpu.SemaphoreType.DMA,
        ],
    )


def _make_deg_kernel(n_pad, rows_per_worker):
    n_chunks = rows_per_worker // CR
    rpt = n_pad // NS

    def body(dst_hbm, out_hbm, deg_sh, zbuf, ones_v, dst_v, gsem):
        del gsem
        c = lax.axis_index("c")
        s = lax.axis_index("s")
        wid = s * NC + c

        zv = jnp.zeros((16,), jnp.float32)
        ov = jnp.ones((16,), jnp.float32)
        for j in range(LANES // 16):
            zbuf[pl.ds(j * 16, 16)] = zv
            ones_v[pl.ds(j * 16, 16)] = ov
        rbase = s * rpt
        for r in range(rpt // LANES):
            pltpu.sync_copy(zbuf, deg_sh.at[pl.ds(rbase + r * LANES, LANES)])
        plsc.subcore_barrier()

        row0 = wid * rows_per_worker

        def chunk(i, carry):
            base = row0 + i * CR
            pltpu.sync_copy(dst_hbm.at[pl.ds(base, CR)], dst_v)
            for r in range(CR):
                pltpu.sync_copy(ones_v, deg_sh.at[dst_v.at[r]], add=True)
            return carry

        lax.fori_loop(0, n_chunks, chunk, 0)
        plsc.subcore_barrier()

        pltpu.sync_copy(deg_sh.at[pl.ds(rbase, rpt)],
                        out_hbm.at[c].at[pl.ds(rbase, rpt)])

    return pl.kernel(
        body,
        out_type=jax.ShapeDtypeStruct((NC, n_pad), jnp.float32),
        mesh=_sc_mesh(),
        scratch_types=[
            pltpu.VMEM_SHARED((n_pad,), jnp.float32),
            pltpu.VMEM((LANES,), jnp.float32),
            pltpu.VMEM((LANES,), jnp.float32),
            pltpu.VMEM((CR, LANES), jnp.int32),
            pltpu.SemaphoreType.DMA,
        ],
    )


def _tc_layer(parts, deg0, deg1, x, wl, wr, b, relu):
    n, d = x.shape
    rb = 1000
    grid = (n // rb,)

    def body(a0, a1, d0, d1, xr, wlr, wrr, br, out):
        deg = jnp.maximum(d0[...] + d1[...], 1.0)
        m = (a0[0] + a1[0]) / deg
        h = jnp.dot(m, wlr[...], preferred_element_type=jnp.float32)
        h = h + jnp.dot(xr[...], wrr[...], preferred_element_type=jnp.float32)
        h = h + br[...][None, :]
        if relu:
            h = jnp.maximum(h, 0.0)
        out[...] = h + xr[...]

    return pl.pallas_call(
        body,
        grid=grid,
        in_specs=[
            pl.BlockSpec((1, rb, d), lambda i: (0, i, 0)),
            pl.BlockSpec((1, rb, d), lambda i: (1, i, 0)),
            pl.BlockSpec((rb, 1), lambda i: (i, 0)),
            pl.BlockSpec((rb, 1), lambda i: (i, 0)),
            pl.BlockSpec((rb, d), lambda i: (i, 0)),
            pl.BlockSpec((d, d), lambda i: (0, 0)),
            pl.BlockSpec((d, d), lambda i: (0, 0)),
            pl.BlockSpec((d,), lambda i: (0,)),
        ],
        out_specs=pl.BlockSpec((rb, d), lambda i: (i, 0)),
        out_shape=jax.ShapeDtypeStruct((n, d), jnp.float32),
    )(parts, parts, deg0, deg1, x, wl, wr, b)


def kernel(x, edge_index, W_l0, W_r0, b0, W_l1, W_r1, b1, W_l2, W_r2, b2):
    n, d = x.shape
    e = edge_index.shape[1]

    n_pad = ((n + 16 * NS - 1) // (16 * NS)) * (16 * NS)
    chunk_edges = NW * LANES * CR
    e_pad = ((e + chunk_edges - 1) // chunk_edges) * chunk_edges
    epw = e_pad // NW
    rows_per_worker = epw // LANES
    pad = e_pad - e

    src = edge_index[0]
    dst = edge_index[1]
    if pad:
        pad_src = (jnp.arange(pad, dtype=jnp.int32) * 97) % n
        pad_dst = n + (jnp.arange(pad, dtype=jnp.int32) % (n_pad - n))
        src = jnp.concatenate([src, pad_src])
        dst = jnp.concatenate([dst, pad_dst.astype(jnp.int32)])
    src2d = src.reshape(-1, LANES)
    dst2d = dst.reshape(-1, LANES)

    deg_k = _make_deg_kernel(n_pad, rows_per_worker)
    agg_k = _make_agg_kernel(n_pad, d, rows_per_worker)

    deg_parts = deg_k(dst2d)
    x, deg_parts = lax.optimization_barrier((x, deg_parts))
    deg0 = lax.slice(deg_parts[0], (0,), (n,)).reshape(n, 1)
    deg1 = lax.slice(deg_parts[1], (0,), (n,)).reshape(n, 1)

    layers = [(W_l0, W_r0, b0), (W_l1, W_r1, b1), (W_l2, W_r2, b2)]
    for i, (wl, wr, b) in enumerate(layers):
        parts = agg_k(x, src2d, dst2d)
        parts, x = lax.optimization_barrier((parts, x))
        parts = lax.slice(parts, (0, 0, 0), (NC, n, d))
        x = _tc_layer(parts, deg0, deg1, x, wl, wr, b, relu=(i != 2))
        x, src2d, dst2d = lax.optimization_barrier((x, src2d, dst2d))
    return x
